# Initial kernel scaffold; baseline (speedup 1.0000x reference)
import jax
import jax.numpy as jnp
from jax.experimental import pallas as pl


def kernel(z, batch, W1, b1, W2, b2):
    raise NotImplementedError("write your pallas kernel here")


# speedup vs baseline: 6.2313x; 6.2313x over previous
import functools

import jax
import jax.numpy as jnp
from jax.experimental import pallas as pl
from jax.experimental.pallas import tpu as pltpu

N = 100000
D = 128
G = 512
BA = 4000
NB = N // BA


def _pass_a(z_ref, w1_ref, b1_ref, w2_ref, b2_ref, a_ref, m_ref):
    i = pl.program_id(0)
    h = jnp.tanh(
        jax.lax.dot_general(z_ref[...], w1_ref[...],
                            (((1,), (1,)), ((), ())),
                            preferred_element_type=jnp.float32)
        + b1_ref[...][None, :])
    al = jax.lax.dot_general(h, w2_ref[...], (((1,), (1,)), ((), ())),
                             preferred_element_type=jnp.float32)
    a_row = al[:, 0] + b2_ref[0, 0]
    a_ref[0, 0, :] = a_row

    @pl.when(i == 0)
    def _():
        m_ref[0, 0] = -jnp.inf

    m_ref[0, 0] = jnp.maximum(m_ref[0, 0], jnp.max(a_row))


def _pass_b(a_ref, m_ref, z_ref, batch_ref, out_ref, accz_ref, accs_ref):
    i = pl.program_id(0)

    @pl.when(i == 0)
    def _():
        accz_ref[...] = jnp.zeros_like(accz_ref)
        accs_ref[...] = jnp.zeros_like(accs_ref)

    e = jnp.exp(a_ref[0, 0, :] - m_ref[0, 0])
    seg = batch_ref[0, 0, :]
    p = (seg[:, None] == jax.lax.iota(jnp.int32, G)[None, :]).astype(
        jnp.float32)
    x = z_ref[...] * e[:, None]
    accz_ref[...] += jax.lax.dot_general(
        p, x, (((0,), (0,)), ((), ())), preferred_element_type=jnp.float32)
    accs_ref[...] += jax.lax.dot_general(
        p, jnp.broadcast_to(e[:, None], (BA, 8)),
        (((0,), (0,)), ((), ())), preferred_element_type=jnp.float32)

    @pl.when(i == NB - 1)
    def _():
        out_ref[...] = accz_ref[...] / (accs_ref[:, 0:1] + 1e-8)


def kernel(z, batch, W1, b1, W2, b2):
    batch = batch.astype(jnp.int32)
    b2_2d = b2.reshape(1, 1)

    a2d, m = pl.pallas_call(
        _pass_a,
        grid=(NB,),
        in_specs=[
            pl.BlockSpec((BA, D), lambda i: (i, 0)),
            pl.BlockSpec((D, D), lambda i: (0, 0)),
            pl.BlockSpec((D,), lambda i: (0,)),
            pl.BlockSpec((1, D), lambda i: (0, 0)),
            pl.BlockSpec((1, 1), lambda i: (0, 0), memory_space=pltpu.SMEM),
        ],
        out_specs=[
            pl.BlockSpec((1, 1, BA), lambda i: (i, 0, 0)),
            pl.BlockSpec((1, 1), lambda i: (0, 0), memory_space=pltpu.SMEM),
        ],
        out_shape=[
            jax.ShapeDtypeStruct((NB, 1, BA), jnp.float32),
            jax.ShapeDtypeStruct((1, 1), jnp.float32),
        ],
    )(z, W1, b1, W2, b2_2d)

    a3d = a2d
    batch3d = batch.reshape(NB, 1, BA)

    out = pl.pallas_call(
        _pass_b,
        grid=(NB,),
        in_specs=[
            pl.BlockSpec((1, 1, BA), lambda i: (i, 0, 0)),
            pl.BlockSpec((1, 1), lambda i: (0, 0), memory_space=pltpu.SMEM),
            pl.BlockSpec((BA, D), lambda i: (i, 0)),
            pl.BlockSpec((1, 1, BA), lambda i: (i, 0, 0)),
        ],
        out_specs=pl.BlockSpec((G, D), lambda i: (0, 0)),
        out_shape=jax.ShapeDtypeStruct((G, D), jnp.float32),
        scratch_shapes=[
            pltpu.VMEM((G, D), jnp.float32),
            pltpu.VMEM((G, 8), jnp.float32),
        ],
    )(a3d, m, z, batch3d)
    return out
